# baseline (speedup 1.0000x reference)
import functools

import jax
import jax.numpy as jnp
import numpy as np
from jax import lax
from jax.experimental import pallas as pl


_N = 4096
_n = np.arange(_N)
_ci = _n // 256
_cj = (_n // 16) % 16
_ck = _n % 16

_p8 = np.zeros((_N, 8), np.float32)
_p8[:, 0] = _ci / 15.0 - 0.5
_p8[:, 1] = _cj / 15.0 - 0.5
_p8[:, 2] = _ck / 15.0 - 0.5

_OHiT = np.zeros((16, _N), np.float32)
_OHiT[_ci, _n] = 1.0
_OHt = np.zeros((_N, 32), np.float32)
_OHt[_n, _ci + _cj] = 1.0
_OHtT = np.ascontiguousarray(_OHt.T)

_l16 = np.arange(16)[None, :]
_mDlo = (_l16 < _ci[:, None]).astype(np.float32)
_mDhi = (_l16 > _ci[:, None]).astype(np.float32)
_mHlo = (_l16 < _cj[:, None]).astype(np.float32)
_mHhi = (_l16 > _cj[:, None]).astype(np.float32)
_mask48 = np.concatenate(
    [_mDlo + _mDhi, _mHlo + _mHhi, np.ones((_N, 16), np.float32)], axis=1)

_GM = (np.arange(128)[:, None] // 4 == np.arange(128)[None, :] // 4).astype(
    np.float32)


def _body(x2_ref, wcat_ref, bcat_ref, p8_ref, wgeo_ref, bgeo_ref,
          ohit_ref, ohtt_ref, oht_ref, mdlo_ref, mdhi_ref, mhlo_ref,
          mhhi_ref, mask48_ref, rw_ref, rb_ref, gnw_ref, gnb_ref,
          gm_ref, out_ref):
    f32 = jnp.float32
    x2 = x2_ref[...]

    tpg = lax.dot_general(x2, wcat_ref[...], (((0,), (0,)), ((), ())),
                          preferred_element_type=f32) + bcat_ref[...]
    theta = tpg[:, 0:64]
    phi = tpg[:, 64:128]
    gfeat = tpg[:, 128:192]
    ptab = jnp.dot(p8_ref[...], wgeo_ref[...],
                   preferred_element_type=f32) + bgeo_ref[...]
    pth = ptab[:, 0:32]
    pph = ptab[:, 32:64]

    def line_dots(a, b, c):
        dnum = (((2,), (2,)), ((0,), (0,)))
        a3 = a.reshape(16, 256, c)
        b3 = b.reshape(16, 256, c)
        aD = jnp.transpose(a3, (1, 0, 2))
        bD = jnp.transpose(b3, (1, 0, 2))
        ld3 = lax.dot_general(aD, bD, dnum, preferred_element_type=f32)
        ld = jnp.transpose(ld3, (1, 0, 2)).reshape(_N, 16)

        a4 = a.reshape(16, 16, 16, c)
        b4 = b.reshape(16, 16, 16, c)
        aH = jnp.transpose(a4, (0, 2, 1, 3)).reshape(256, 16, c)
        bH = jnp.transpose(b4, (0, 2, 1, 3)).reshape(256, 16, c)
        lh3 = lax.dot_general(aH, bH, dnum, preferred_element_type=f32)
        lh = jnp.transpose(lh3.reshape(16, 16, 16, 16),
                           (0, 2, 1, 3)).reshape(_N, 16)

        aW = a.reshape(256, 16, c)
        bW = b.reshape(256, 16, c)
        lw3 = lax.dot_general(aW, bW, dnum, preferred_element_type=f32)
        lw = lw3.reshape(_N, 16)
        return ld, lh, lw

    fD, fH, fW = line_dots(theta, phi, 64)
    pD, pH, pW = line_dots(pth, pph, 32)

    mdlo = mdlo_ref[...]
    mdhi = mdhi_ref[...]
    mhlo = mhlo_ref[...]
    mhhi = mhhi_ref[...]
    ohit = ohit_ref[...]
    ohtt = ohtt_ref[...]
    oht = oht_ref[...]

    def slot_inv_norms(ld, lh, lw):
        ld2 = ld * ld
        lh2 = lh * lh
        lw2 = lw * lw
        s = jnp.zeros((1, 64), f32)
        sd_lo = jnp.sum(ld2 * mdlo, axis=0, keepdims=True)
        sd_hi = jnp.sum(ld2 * mdhi, axis=0, keepdims=True)
        s = s + jnp.pad(sd_lo, ((0, 0), (0, 48)))
        s = s + jnp.pad(sd_hi, ((0, 0), (30, 18)))
        th_lo = jnp.dot(ohit, lh2 * mhlo,
                        preferred_element_type=f32)
        th_hi = jnp.dot(ohit, lh2 * mhhi,
                        preferred_element_type=f32)
        aw = jnp.dot(ohtt, lw2, preferred_element_type=f32)
        for t in range(16):
            s = s + jnp.pad(th_lo[t:t + 1, :], ((0, 0), (t, 48 - t)))
            s = s + jnp.pad(th_hi[t:t + 1, :], ((0, 0), (t + 15, 33 - t)))
        for t in range(31):
            s = s + jnp.pad(aw[t:t + 1, :], ((0, 0), (t, 48 - t)))
        return 1.0 / (1e-6 + jnp.sqrt(s))

    vf = slot_inv_norms(fD, fH, fW)
    vp = slot_inv_norms(pD, pH, pW)

    def gather_inv(v):
        h0 = jnp.concatenate([v[:, i:i + 16] for i in range(16)], axis=0)
        h15 = jnp.concatenate([v[:, i + 15:i + 31] for i in range(16)], axis=0)
        h31 = jnp.concatenate(
            [v[:, t:t + 16] for t in range(31)] + [jnp.zeros((1, 16), f32)],
            axis=0)
        v_d = (mdlo * jnp.broadcast_to(v[:, 0:16], (_N, 16))
               + mdhi * jnp.broadcast_to(v[:, 30:46], (_N, 16)))
        by_i = lambda hh: jnp.broadcast_to(
            hh.reshape(16, 1, 16), (16, 256, 16)).reshape(_N, 16)
        v_h = mhlo * by_i(h0) + mhhi * by_i(h15)
        v_w = jnp.dot(oht, h31, preferred_element_type=f32)
        return v_d, v_h, v_w

    vfD, vfH, vfW = gather_inv(vf)
    vpD, vpH, vpW = gather_inv(vp)

    def logit(fv, pv, nf, np_):
        return fv * nf + jnp.maximum(pv * np_, 0.0)

    lD = logit(fD, pD, vfD, vpD)
    lH = logit(fH, pH, vfH, vpH)
    lW = logit(fW, pW, vfW, vpW)
    lg = jnp.concatenate([lD, lH, lW], axis=1)
    ex = jnp.exp(jnp.minimum(lg, 3.0)) * mask48_ref[...]
    wsm = ex / jnp.sum(ex, axis=1, keepdims=True)

    wD = wsm[:, 0:16]
    wH = wsm[:, 16:32]
    wW = wsm[:, 32:48]
    dny = (((2,), (1,)), ((0,), (0,)))

    g3 = gfeat.reshape(16, 256, 64)
    wD3 = jnp.transpose(wD.reshape(16, 256, 16), (1, 0, 2))
    gD = jnp.transpose(g3, (1, 0, 2))
    yD3 = lax.dot_general(wD3, gD, dny, preferred_element_type=f32)
    yD = jnp.transpose(yD3, (1, 0, 2)).reshape(_N, 64)

    g4 = gfeat.reshape(16, 16, 16, 64)
    wH3 = jnp.transpose(wH.reshape(16, 16, 16, 16),
                        (0, 2, 1, 3)).reshape(256, 16, 16)
    gH = jnp.transpose(g4, (0, 2, 1, 3)).reshape(256, 16, 64)
    yH3 = lax.dot_general(wH3, gH, dny, preferred_element_type=f32)
    yH = jnp.transpose(yH3.reshape(16, 16, 16, 64),
                       (0, 2, 1, 3)).reshape(_N, 64)

    wW3 = wW.reshape(256, 16, 16)
    gW = gfeat.reshape(256, 16, 64)
    yW3 = lax.dot_general(wW3, gW, dny, preferred_element_type=f32)
    yW = yW3.reshape(_N, 64)

    y = yD + yH + yW

    cross_cm = lax.dot_general(rw_ref[...], y, (((1,), (1,)), ((), ())),
                               preferred_element_type=f32) + rb_ref[...]
    hn = x2 + cross_cm
    m1 = jnp.sum(hn, axis=1, keepdims=True)
    m2 = jnp.sum(hn * hn, axis=1, keepdims=True)
    g1 = jnp.dot(gm_ref[...], m1, preferred_element_type=f32)
    g2 = jnp.dot(gm_ref[...], m2, preferred_element_type=f32)
    cnt = jnp.float32(4.0 * _N)
    mu = g1 / cnt
    var = g2 / cnt - mu * mu
    inv = lax.rsqrt(var + 1e-5)
    out_ref[...] = (hn - mu) * inv * gnw_ref[...] + gnb_ref[...]


@functools.partial(jax.jit, static_argnames=("interpret",))
def _run(x2, wcat, bcat, wgeo, bgeo, rw, rb, gnw, gnb, interpret=False):
    consts = (jnp.asarray(_p8), jnp.asarray(_OHiT), jnp.asarray(_OHtT),
              jnp.asarray(_OHt), jnp.asarray(_mDlo), jnp.asarray(_mDhi),
              jnp.asarray(_mHlo), jnp.asarray(_mHhi), jnp.asarray(_mask48),
              jnp.asarray(_GM))
    p8, ohit, ohtt, oht, mdlo, mdhi, mhlo, mhhi, mask48, gm = consts
    return pl.pallas_call(
        _body,
        out_shape=jax.ShapeDtypeStruct((128, _N), jnp.float32),
        interpret=interpret,
    )(x2, wcat, bcat, p8, wgeo, bgeo, ohit, ohtt, oht,
      mdlo, mdhi, mhlo, mhhi, mask48, rw, rb, gnw, gnb, gm)


def kernel(x, G_w, G_b, theta_w, theta_b, phi_w, phi_b, r_w, r_b,
           geo_theta_w, geo_theta_b, geo_phi_w, geo_phi_b, gn_w, gn_b, nbr,
           interpret=False):
    del nbr
    B, C, D, H, W = x.shape
    x2 = x.reshape(C, D * H * W)
    wcat = jnp.concatenate([theta_w, phi_w, G_w], axis=0).T
    bcat = jnp.concatenate([theta_b, phi_b, G_b]).reshape(1, 192)
    wgeo = jnp.pad(jnp.concatenate([geo_theta_w, geo_phi_w], axis=0).T,
                   ((0, 5), (0, 0)))
    bgeo = jnp.concatenate([geo_theta_b, geo_phi_b]).reshape(1, 64)
    out = _run(x2, wcat, bcat, wgeo, bgeo, r_w, r_b.reshape(128, 1),
               gn_w.reshape(128, 1), gn_b.reshape(128, 1),
               interpret=interpret)
    return out.reshape(B, C, D, H, W)


# speedup vs baseline: 1.5208x; 1.5208x over previous
import functools

import jax
import jax.numpy as jnp
import numpy as np
from jax import lax
from jax.experimental import pallas as pl


_N = 4096
_n = np.arange(_N)
_ci = _n // 256
_cj = (_n // 16) % 16
_ck = _n % 16

_p8 = np.zeros((_N, 8), np.float32)
_p8[:, 0] = _ci / 15.0 - 0.5
_p8[:, 1] = _cj / 15.0 - 0.5
_p8[:, 2] = _ck / 15.0 - 0.5

_OHiT = np.zeros((16, _N), np.float32)
_OHiT[_ci, _n] = 1.0
_OHt = np.zeros((_N, 32), np.float32)
_OHt[_n, _ci + _cj] = 1.0
_OHtT = np.ascontiguousarray(_OHt.T)

_l16 = np.arange(16)[None, :]
_mDlo = (_l16 < _ci[:, None]).astype(np.float32)
_mDhi = (_l16 > _ci[:, None]).astype(np.float32)
_mHlo = (_l16 < _cj[:, None]).astype(np.float32)
_mHhi = (_l16 > _cj[:, None]).astype(np.float32)
_mDlo2 = np.tile(_mDlo, (1, 2))
_mDhi2 = np.tile(_mDhi, (1, 2))
_mHlo2 = np.tile(_mHlo, (1, 2))
_mHhi2 = np.tile(_mHhi, (1, 2))
_mask48 = np.concatenate(
    [_mDlo + _mDhi, _mHlo + _mHhi, np.ones((_N, 16), np.float32)], axis=1)

_GM = (np.arange(128)[:, None] // 4 == np.arange(128)[None, :] // 4).astype(
    np.float32)


def _body(h2_ref, wcat_ref, bcat_ref, p8_ref, wgeo_ref, bgeo_ref,
          ohit_ref, ohtt_ref, oht_ref, mdlo_ref, mdhi_ref, mhlo_ref,
          mhhi_ref, mask48_ref, rwt_ref, rb_ref, gnw_ref, gnb_ref,
          gm_ref, out_ref):
    f32 = jnp.float32
    h2 = h2_ref[...]

    tpg = jnp.dot(h2, wcat_ref[...],
                  preferred_element_type=f32) + bcat_ref[...]
    theta = tpg[:, 0:64]
    phi = tpg[:, 64:128]
    gfeat = tpg[:, 128:192]
    ptab = jnp.dot(p8_ref[...], wgeo_ref[...],
                   preferred_element_type=f32) + bgeo_ref[...]
    pth = ptab[:, 0:32]
    pph = ptab[:, 32:64]

    def line_dots(a, b, c):
        dnum = (((2,), (2,)), ((0,), (0,)))
        a3 = a.reshape(16, 256, c)
        b3 = b.reshape(16, 256, c)
        aD = jnp.transpose(a3, (1, 0, 2))
        bD = jnp.transpose(b3, (1, 0, 2))
        ld3 = lax.dot_general(aD, bD, dnum, preferred_element_type=f32)
        ld = jnp.transpose(ld3, (1, 0, 2)).reshape(_N, 16)

        a4 = a.reshape(16, 16, 16, c)
        b4 = b.reshape(16, 16, 16, c)
        aH = jnp.transpose(a4, (0, 2, 1, 3)).reshape(256, 16, c)
        bH = jnp.transpose(b4, (0, 2, 1, 3)).reshape(256, 16, c)
        lh3 = lax.dot_general(aH, bH, dnum, preferred_element_type=f32)
        lh = jnp.transpose(lh3.reshape(16, 16, 16, 16),
                           (0, 2, 1, 3)).reshape(_N, 16)

        aW = a.reshape(256, 16, c)
        bW = b.reshape(256, 16, c)
        lw3 = lax.dot_general(aW, bW, dnum, preferred_element_type=f32)
        lw = lw3.reshape(_N, 16)
        return ld, lh, lw

    fD, fH, fW = line_dots(theta, phi, 64)
    pD, pH, pW = line_dots(pth, pph, 32)
    fpD = jnp.concatenate([fD, pD], axis=1)
    fpH = jnp.concatenate([fH, pH], axis=1)
    fpW = jnp.concatenate([fW, pW], axis=1)

    mdlo2 = mdlo_ref[...]
    mdhi2 = mdhi_ref[...]
    mhlo2 = mhlo_ref[...]
    mhhi2 = mhhi_ref[...]
    ohit = ohit_ref[...]
    ohtt = ohtt_ref[...]
    oht = oht_ref[...]

    sqD = fpD * fpD
    sqH = fpH * fpH
    sqW = fpW * fpW
    sd_lo = jnp.sum(sqD * mdlo2, axis=0, keepdims=True)
    sd_hi = jnp.sum(sqD * mdhi2, axis=0, keepdims=True)
    thx = jnp.dot(ohit, jnp.concatenate([sqH * mhlo2, sqH * mhhi2], axis=1),
                  preferred_element_type=f32)
    aw = jnp.dot(ohtt, sqW, preferred_element_type=f32)
    sf = jnp.pad(sd_lo[:, 0:16], ((0, 0), (0, 48)))
    sp = jnp.pad(sd_lo[:, 16:32], ((0, 0), (0, 48)))
    sf = sf + jnp.pad(sd_hi[:, 0:16], ((0, 0), (30, 18)))
    sp = sp + jnp.pad(sd_hi[:, 16:32], ((0, 0), (30, 18)))
    for t in range(16):
        sf = sf + jnp.pad(thx[t:t + 1, 0:16], ((0, 0), (t, 48 - t)))
        sp = sp + jnp.pad(thx[t:t + 1, 16:32], ((0, 0), (t, 48 - t)))
        sf = sf + jnp.pad(thx[t:t + 1, 32:48], ((0, 0), (t + 15, 33 - t)))
        sp = sp + jnp.pad(thx[t:t + 1, 48:64], ((0, 0), (t + 15, 33 - t)))
    for t in range(31):
        sf = sf + jnp.pad(aw[t:t + 1, 0:16], ((0, 0), (t, 48 - t)))
        sp = sp + jnp.pad(aw[t:t + 1, 16:32], ((0, 0), (t, 48 - t)))
    vf = 1.0 / (1e-6 + jnp.sqrt(sf))
    vp = 1.0 / (1e-6 + jnp.sqrt(sp))

    def hank(v, off, rows):
        return jnp.concatenate([v[:, t + off:t + off + 16]
                                for t in range(rows)], axis=0)

    h0 = jnp.concatenate([hank(vf, 0, 16), hank(vp, 0, 16)], axis=1)
    h15 = jnp.concatenate([hank(vf, 15, 16), hank(vp, 15, 16)], axis=1)
    h31 = jnp.concatenate(
        [jnp.concatenate([hank(vf, 0, 31), jnp.zeros((1, 16), f32)], axis=0),
         jnp.concatenate([hank(vp, 0, 31), jnp.zeros((1, 16), f32)], axis=0)],
        axis=1)
    vlo = jnp.concatenate([vf[:, 0:16], vp[:, 0:16]], axis=1)
    vhi = jnp.concatenate([vf[:, 30:46], vp[:, 30:46]], axis=1)
    v_d = (mdlo2 * jnp.broadcast_to(vlo, (_N, 32))
           + mdhi2 * jnp.broadcast_to(vhi, (_N, 32)))
    by_i = lambda hh: jnp.broadcast_to(
        hh.reshape(16, 1, 32), (16, 256, 32)).reshape(_N, 32)
    v_h = mhlo2 * by_i(h0) + mhhi2 * by_i(h15)
    v_w = jnp.dot(oht, h31, preferred_element_type=f32)

    nD = fpD * v_d
    nH = fpH * v_h
    nW = fpW * v_w
    lD = nD[:, 0:16] + jnp.maximum(nD[:, 16:32], 0.0)
    lH = nH[:, 0:16] + jnp.maximum(nH[:, 16:32], 0.0)
    lW = nW[:, 0:16] + jnp.maximum(nW[:, 16:32], 0.0)
    lg = jnp.concatenate([lD, lH, lW], axis=1)
    ex = jnp.exp(jnp.minimum(lg, 3.0)) * mask48_ref[...]
    wsm = ex / jnp.sum(ex, axis=1, keepdims=True)

    wD = wsm[:, 0:16]
    wH = wsm[:, 16:32]
    wW = wsm[:, 32:48]
    dny = (((2,), (1,)), ((0,), (0,)))

    g3 = gfeat.reshape(16, 256, 64)
    wD3 = jnp.transpose(wD.reshape(16, 256, 16), (1, 0, 2))
    gD = jnp.transpose(g3, (1, 0, 2))
    yD3 = lax.dot_general(wD3, gD, dny, preferred_element_type=f32)
    yD = jnp.transpose(yD3, (1, 0, 2)).reshape(_N, 64)

    g4 = gfeat.reshape(16, 16, 16, 64)
    wH3 = jnp.transpose(wH.reshape(16, 16, 16, 16),
                        (0, 2, 1, 3)).reshape(256, 16, 16)
    gH = jnp.transpose(g4, (0, 2, 1, 3)).reshape(256, 16, 64)
    yH3 = lax.dot_general(wH3, gH, dny, preferred_element_type=f32)
    yH = jnp.transpose(yH3.reshape(16, 16, 16, 64),
                       (0, 2, 1, 3)).reshape(_N, 64)

    wW3 = wW.reshape(256, 16, 16)
    gW = gfeat.reshape(256, 16, 64)
    yW3 = lax.dot_general(wW3, gW, dny, preferred_element_type=f32)
    yW = yW3.reshape(_N, 64)

    y = yD + yH + yW

    cross = jnp.dot(y, rwt_ref[...],
                    preferred_element_type=f32) + rb_ref[...]
    hn = h2 + cross
    m1 = jnp.sum(hn, axis=0, keepdims=True)
    m2 = jnp.sum(hn * hn, axis=0, keepdims=True)
    g1 = jnp.dot(m1, gm_ref[...], preferred_element_type=f32)
    g2 = jnp.dot(m2, gm_ref[...], preferred_element_type=f32)
    cnt = jnp.float32(4.0 * _N)
    mu = g1 / cnt
    var = g2 / cnt - mu * mu
    inv = lax.rsqrt(var + 1e-5)
    out_ref[...] = (hn - mu) * inv * gnw_ref[...] + gnb_ref[...]


@functools.partial(jax.jit, static_argnames=("interpret",))
def _run(h2, wcat, bcat, wgeo, bgeo, rwt, rb, gnw, gnb, interpret=False):
    consts = (jnp.asarray(_p8), jnp.asarray(_OHiT), jnp.asarray(_OHtT),
              jnp.asarray(_OHt), jnp.asarray(_mDlo2), jnp.asarray(_mDhi2),
              jnp.asarray(_mHlo2), jnp.asarray(_mHhi2), jnp.asarray(_mask48),
              jnp.asarray(_GM))
    p8, ohit, ohtt, oht, mdlo, mdhi, mhlo, mhhi, mask48, gm = consts
    return pl.pallas_call(
        _body,
        out_shape=jax.ShapeDtypeStruct((_N, 128), jnp.float32),
        interpret=interpret,
    )(h2, wcat, bcat, p8, wgeo, bgeo, ohit, ohtt, oht,
      mdlo, mdhi, mhlo, mhhi, mask48, rwt, rb, gnw, gnb, gm)


def kernel(x, G_w, G_b, theta_w, theta_b, phi_w, phi_b, r_w, r_b,
           geo_theta_w, geo_theta_b, geo_phi_w, geo_phi_b, gn_w, gn_b, nbr,
           interpret=False):
    del nbr
    B, C, D, H, W = x.shape
    h2 = jnp.transpose(x.reshape(C, D * H * W))
    wcat = jnp.concatenate([theta_w, phi_w, G_w], axis=0).T
    bcat = jnp.concatenate([theta_b, phi_b, G_b]).reshape(1, 192)
    wgeo = jnp.pad(jnp.concatenate([geo_theta_w, geo_phi_w], axis=0).T,
                   ((0, 5), (0, 0)))
    bgeo = jnp.concatenate([geo_theta_b, geo_phi_b]).reshape(1, 64)
    out = _run(h2, wcat, bcat, wgeo, bgeo, r_w.T, r_b.reshape(1, 128),
               gn_w.reshape(1, 128), gn_b.reshape(1, 128),
               interpret=interpret)
    return jnp.transpose(out).reshape(B, C, D, H, W)
